# Initial kernel scaffold; baseline (speedup 1.0000x reference)
import jax
import jax.numpy as jnp
from jax.experimental import pallas as pl


def kernel(sample, entity_embedding, relation_embedding_head, relation_embedding_tail):
    raise NotImplementedError("write your pallas kernel here")


# speedup vs baseline: 1.3638x; 1.3638x over previous
import functools

import jax
import jax.numpy as jnp
from jax import lax
from jax.experimental import pallas as pl
from jax.experimental.pallas import tpu as pltpu
from jax.experimental.pallas import tpu_sc as plsc

D = 128
GAMMA = 6.0
L = 16
NC = 2
NS = 16
NW = NC * NS
CHUNK = 64
NORM_EPS2 = 1e-24


def _rsqrt(x):
    i = lax.bitcast_convert_type(x, jnp.int32)
    i = jnp.int32(0x5F3759DF) - lax.shift_right_logical(i, 1)
    y = lax.bitcast_convert_type(i, jnp.float32)
    for _ in range(3):
        y = y * (1.5 - 0.5 * x * y * y)
    return y


_GATHER_DNUMS = lax.GatherDimensionNumbers(
    offset_dims=(), collapsed_slice_dims=(0,), start_index_map=(0,))


def _lane_shuffle(v, idx):
    return lax.gather(v, idx.reshape(L, 1), _GATHER_DNUMS, (1,),
                      mode=lax.GatherScatterMode.PROMISE_IN_BOUNDS)


def _lane_sum(v):
    for s in (8, 4, 2, 1):
        v = v + _lane_shuffle(v, lax.iota(jnp.int32, L) ^ s)
    return v


@functools.lru_cache(maxsize=None)
def _make_sc_kernel(B):
    assert B % (NW * CHUNK) == 0
    npw = B // NW
    nchunk = npw // CHUNK
    mesh = plsc.VectorSubcoreMesh(core_axis_name="c", subcore_axis_name="s")

    @functools.partial(
        pl.kernel,
        mesh=mesh,
        out_type=jax.ShapeDtypeStruct((B // L, L), jnp.float32),
        scratch_types=[
            pltpu.VMEM((npw,), jnp.int32),
            pltpu.VMEM((npw,), jnp.int32),
            pltpu.VMEM((npw,), jnp.int32),
            pltpu.VMEM((2, CHUNK, D), jnp.float32),
            pltpu.VMEM((2, CHUNK, D), jnp.float32),
            pltpu.VMEM((2, CHUNK, D), jnp.float32),
            pltpu.VMEM((2, CHUNK, D), jnp.float32),
            pltpu.VMEM((npw // L, L), jnp.float32),
            pltpu.SemaphoreType.DMA,
            pltpu.SemaphoreType.DMA,
        ],
    )
    def rgcn_sc(hidx_hbm, ridx_hbm, tidx_hbm, ent_hbm, relh_hbm, relt_hbm,
                out_hbm, hidx_v, ridx_v, tidx_v, h_v, rh_v, rt_v, t_v,
                out_v, sem0, sem1):
        wid = lax.axis_index("s") * NC + lax.axis_index("c")
        base = wid * npw

        pltpu.sync_copy(hidx_hbm.at[pl.ds(base, npw)], hidx_v)
        pltpu.sync_copy(ridx_hbm.at[pl.ds(base, npw)], ridx_v)
        pltpu.sync_copy(tidx_hbm.at[pl.ds(base, npw)], tidx_v)

        sems = (sem0, sem1)

        def fire(c, bi):
            o = c * CHUNK
            s = sems[bi]
            hi = hidx_v.at[pl.ds(o, CHUNK)]
            ri = ridx_v.at[pl.ds(o, CHUNK)]
            ti = tidx_v.at[pl.ds(o, CHUNK)]
            return [
                pltpu.async_copy(ent_hbm.at[hi], h_v.at[bi], s),
                pltpu.async_copy(relh_hbm.at[ri], rh_v.at[bi], s),
                pltpu.async_copy(relt_hbm.at[ri], rt_v.at[bi], s),
                pltpu.async_copy(ent_hbm.at[ti], t_v.at[bi], s),
            ]

        lane_iota = lax.iota(jnp.int32, L)

        def chunk_compute(c, bi):
            obase = c * CHUNK

            def body(i, merged):
                hs = [h_v[bi, i, pl.ds(k * L, L)] for k in range(D // L)]
                ts = [t_v[bi, i, pl.ds(k * L, L)] for k in range(D // L)]
                ssh = hs[0] * hs[0]
                sst = ts[0] * ts[0]
                for k in range(1, D // L):
                    ssh = ssh + hs[k] * hs[k]
                    sst = sst + ts[k] * ts[k]
                ihv = _rsqrt(jnp.maximum(_lane_sum(ssh), NORM_EPS2))
                itv = _rsqrt(jnp.maximum(_lane_sum(sst), NORM_EPS2))
                acc = None
                for k in range(D // L):
                    rhk = rh_v[bi, i, pl.ds(k * L, L)]
                    rtk = rt_v[bi, i, pl.ds(k * L, L)]
                    term = jnp.abs(hs[k] * ihv * rhk - ts[k] * itv * rtk)
                    acc = term if acc is None else acc + term
                score = GAMMA - _lane_sum(acc)
                j = lax.bitwise_and(i, L - 1)
                merged = jnp.where(lane_iota == j, score, merged)

                @pl.when(j == L - 1)
                def _():
                    out_v[lax.shift_right_logical(obase + i, 4), :] = merged

                return merged

            lax.fori_loop(0, CHUNK, body, jnp.zeros((L,), jnp.float32))

        pending = fire(0, 0)
        for c in range(nchunk):
            nxt = fire(c + 1, (c + 1) & 1) if c + 1 < nchunk else None
            for hnd in pending:
                hnd.wait()
            chunk_compute(c, c & 1)
            pending = nxt

        pltpu.sync_copy(out_v, out_hbm.at[pl.ds(wid * (npw // L), npw // L)])

    return rgcn_sc


def kernel(sample, entity_embedding, relation_embedding_head, relation_embedding_tail):
    h_idx = sample[:, 0]
    r_idx = sample[:, 1]
    t_idx = sample[:, 2]
    out = _make_sc_kernel(sample.shape[0])(
        h_idx, r_idx, t_idx,
        entity_embedding, relation_embedding_head, relation_embedding_tail)
    return out.reshape(sample.shape[0])


# speedup vs baseline: 1.6878x; 1.2376x over previous
import functools

import jax
import jax.numpy as jnp
from jax import lax
from jax.experimental import pallas as pl
from jax.experimental.pallas import tpu as pltpu
from jax.experimental.pallas import tpu_sc as plsc

D = 128
GAMMA = 6.0
L = 16
NC = 2
NS = 16
NW = NC * NS
CHUNK = 64
NORM_EPS2 = 1e-24


def _rsqrt(x):
    i = lax.bitcast_convert_type(x, jnp.int32)
    i = jnp.int32(0x5F3759DF) - lax.shift_right_logical(i, 1)
    y = lax.bitcast_convert_type(i, jnp.float32)
    hx = 0.5 * x
    for _ in range(3):
        y = y * (1.5 - hx * y * y)
    return y


_GATHER_DNUMS = lax.GatherDimensionNumbers(
    offset_dims=(), collapsed_slice_dims=(0,), start_index_map=(0,))


def _lane_shuffle(v, idx):
    return lax.gather(v, idx.reshape(L, 1), _GATHER_DNUMS, (1,),
                      mode=lax.GatherScatterMode.PROMISE_IN_BOUNDS)


def _lane_sum(v):
    for s in (8, 4, 2, 1):
        v = v + _lane_shuffle(v, lax.iota(jnp.int32, L) ^ s)
    return v


@functools.lru_cache(maxsize=None)
def _make_sc_kernel(B):
    assert B % (NW * CHUNK) == 0
    npw = B // NW
    nchunk = npw // CHUNK
    mesh = plsc.VectorSubcoreMesh(core_axis_name="c", subcore_axis_name="s")

    @functools.partial(
        pl.kernel,
        mesh=mesh,
        out_type=jax.ShapeDtypeStruct((B // L, L), jnp.float32),
        scratch_types=[
            pltpu.VMEM((npw,), jnp.int32),
            pltpu.VMEM((npw,), jnp.int32),
            pltpu.VMEM((npw,), jnp.int32),
            pltpu.VMEM((2, CHUNK, D), jnp.float32),
            pltpu.VMEM((2, CHUNK, D), jnp.float32),
            pltpu.VMEM((2, CHUNK, D), jnp.float32),
            pltpu.VMEM((2, CHUNK, D), jnp.float32),
            pltpu.VMEM((npw // L, L), jnp.float32),
            pltpu.SemaphoreType.DMA,
            pltpu.SemaphoreType.DMA,
        ],
    )
    def rgcn_sc(hidx_hbm, ridx_hbm, tidx_hbm, ent_hbm, relh_hbm, relt_hbm,
                out_hbm, hidx_v, ridx_v, tidx_v, h_v, rh_v, rt_v, t_v,
                out_v, sem0, sem1):
        wid = lax.axis_index("s") * NC + lax.axis_index("c")
        base = wid * npw

        pltpu.sync_copy(hidx_hbm.at[pl.ds(base, npw)], hidx_v)
        pltpu.sync_copy(ridx_hbm.at[pl.ds(base, npw)], ridx_v)
        pltpu.sync_copy(tidx_hbm.at[pl.ds(base, npw)], tidx_v)

        sems = (sem0, sem1)

        def fire(c, bi):
            o = c * CHUNK
            s = sems[bi]
            hi = hidx_v.at[pl.ds(o, CHUNK)]
            ri = ridx_v.at[pl.ds(o, CHUNK)]
            ti = tidx_v.at[pl.ds(o, CHUNK)]
            return [
                pltpu.async_copy(ent_hbm.at[hi], h_v.at[bi], s),
                pltpu.async_copy(relh_hbm.at[ri], rh_v.at[bi], s),
                pltpu.async_copy(relt_hbm.at[ri], rt_v.at[bi], s),
                pltpu.async_copy(ent_hbm.at[ti], t_v.at[bi], s),
            ]

        lane_iota = lax.iota(jnp.int32, L)
        nk = D // L

        def chunk_compute(c, bi):
            obase = c * CHUNK

            def norm_phase(i):
                hs = [h_v[bi, i, pl.ds(k * L, L)] for k in range(nk)]
                ts = [t_v[bi, i, pl.ds(k * L, L)] for k in range(nk)]
                ssh = hs[0] * hs[0]
                sst = ts[0] * ts[0]
                for k in range(1, nk):
                    ssh = ssh + hs[k] * hs[k]
                    sst = sst + ts[k] * ts[k]
                ihv = _rsqrt(jnp.maximum(_lane_sum(ssh), NORM_EPS2))
                itv = _rsqrt(jnp.maximum(_lane_sum(sst), NORM_EPS2))
                return hs, ts, ihv, itv

            def score_phase(i, hs, ts, ihv, itv, merged):
                acc = None
                for k in range(nk):
                    rhk = rh_v[bi, i, pl.ds(k * L, L)]
                    rtk = rt_v[bi, i, pl.ds(k * L, L)]
                    term = jnp.abs(hs[k] * ihv * rhk - ts[k] * itv * rtk)
                    acc = term if acc is None else acc + term
                score = GAMMA - _lane_sum(acc)
                j = lax.bitwise_and(i, L - 1)
                merged = jnp.where(lane_iota == j, score, merged)

                @pl.when(j == L - 1)
                def _():
                    out_v[lax.shift_right_logical(obase + i, 4), :] = merged

                return merged

            hs0, ts0, ihv0, itv0 = norm_phase(0)
            carry0 = (*hs0, *ts0, ihv0, itv0, jnp.zeros((L,), jnp.float32))

            def body(i, carry):
                hs, ts = list(carry[:nk]), list(carry[nk:2 * nk])
                ihv, itv, merged = carry[2 * nk], carry[2 * nk + 1], carry[2 * nk + 2]
                nxt = jnp.minimum(i + 1, CHUNK - 1)
                hs1, ts1, ihv1, itv1 = norm_phase(nxt)
                merged = score_phase(i, hs, ts, ihv, itv, merged)
                return (*hs1, *ts1, ihv1, itv1, merged)

            lax.fori_loop(0, CHUNK, body, carry0)

        pending = fire(0, 0)
        for c in range(nchunk):
            nxt = fire(c + 1, (c + 1) & 1) if c + 1 < nchunk else None
            for hnd in pending:
                hnd.wait()
            chunk_compute(c, c & 1)
            pending = nxt

        pltpu.sync_copy(out_v, out_hbm.at[pl.ds(wid * (npw // L), npw // L)])

    return rgcn_sc


def kernel(sample, entity_embedding, relation_embedding_head, relation_embedding_tail):
    h_idx = sample[:, 0]
    r_idx = sample[:, 1]
    t_idx = sample[:, 2]
    out = _make_sc_kernel(sample.shape[0])(
        h_idx, r_idx, t_idx,
        entity_embedding, relation_embedding_head, relation_embedding_tail)
    return out.reshape(sample.shape[0])


# speedup vs baseline: 1.7071x; 1.0115x over previous
import functools

import jax
import jax.numpy as jnp
from jax import lax
from jax.experimental import pallas as pl
from jax.experimental.pallas import tpu as pltpu
from jax.experimental.pallas import tpu_sc as plsc

D = 128
GAMMA = 6.0
L = 16
NC = 2
NS = 16
NW = NC * NS
CHUNK = 64
NORM_EPS2 = 1e-24


def _rsqrt(x):
    i = lax.bitcast_convert_type(x, jnp.int32)
    i = jnp.int32(0x5F3759DF) - lax.shift_right_logical(i, 1)
    y = lax.bitcast_convert_type(i, jnp.float32)
    hx = 0.5 * x
    for _ in range(2):
        y = y * (1.5 - hx * y * y)
    return y


def _tree_sum(vs):
    vs = list(vs)
    while len(vs) > 1:
        nxt = [vs[i] + vs[i + 1] for i in range(0, len(vs) - 1, 2)]
        if len(vs) % 2:
            nxt.append(vs[-1])
        vs = nxt
    return vs[0]


_GATHER_DNUMS = lax.GatherDimensionNumbers(
    offset_dims=(), collapsed_slice_dims=(0,), start_index_map=(0,))


def _lane_shuffle(v, idx):
    return lax.gather(v, idx.reshape(L, 1), _GATHER_DNUMS, (1,),
                      mode=lax.GatherScatterMode.PROMISE_IN_BOUNDS)


def _lane_sum(v):
    for s in (8, 4, 2, 1):
        v = v + _lane_shuffle(v, lax.iota(jnp.int32, L) ^ s)
    return v


@functools.lru_cache(maxsize=None)
def _make_sc_kernel(B):
    assert B % (NW * CHUNK) == 0
    npw = B // NW
    nchunk = npw // CHUNK
    mesh = plsc.VectorSubcoreMesh(core_axis_name="c", subcore_axis_name="s")

    @functools.partial(
        pl.kernel,
        mesh=mesh,
        out_type=jax.ShapeDtypeStruct((B // L, L), jnp.float32),
        scratch_types=[
            pltpu.VMEM((npw,), jnp.int32),
            pltpu.VMEM((npw,), jnp.int32),
            pltpu.VMEM((npw,), jnp.int32),
            pltpu.VMEM((2, CHUNK, D), jnp.float32),
            pltpu.VMEM((2, CHUNK, D), jnp.float32),
            pltpu.VMEM((2, CHUNK, D), jnp.float32),
            pltpu.VMEM((2, CHUNK, D), jnp.float32),
            pltpu.VMEM((npw // L, L), jnp.float32),
            pltpu.SemaphoreType.DMA,
            pltpu.SemaphoreType.DMA,
        ],
    )
    def rgcn_sc(hidx_hbm, ridx_hbm, tidx_hbm, ent_hbm, relh_hbm, relt_hbm,
                out_hbm, hidx_v, ridx_v, tidx_v, h_v, rh_v, rt_v, t_v,
                out_v, sem0, sem1):
        wid = lax.axis_index("s") * NC + lax.axis_index("c")
        base = wid * npw

        pltpu.sync_copy(hidx_hbm.at[pl.ds(base, npw)], hidx_v)
        pltpu.sync_copy(ridx_hbm.at[pl.ds(base, npw)], ridx_v)
        pltpu.sync_copy(tidx_hbm.at[pl.ds(base, npw)], tidx_v)

        sems = (sem0, sem1)

        def fire(c, bi):
            o = c * CHUNK
            s = sems[bi]
            hi = hidx_v.at[pl.ds(o, CHUNK)]
            ri = ridx_v.at[pl.ds(o, CHUNK)]
            ti = tidx_v.at[pl.ds(o, CHUNK)]
            return [
                pltpu.async_copy(ent_hbm.at[hi], h_v.at[bi], s),
                pltpu.async_copy(relh_hbm.at[ri], rh_v.at[bi], s),
                pltpu.async_copy(relt_hbm.at[ri], rt_v.at[bi], s),
                pltpu.async_copy(ent_hbm.at[ti], t_v.at[bi], s),
            ]

        lane_iota = lax.iota(jnp.int32, L)
        nk = D // L

        def chunk_compute(c, bi):
            obase = c * CHUNK

            def norm_phase(i):
                hs = [h_v[bi, i, pl.ds(k * L, L)] for k in range(nk)]
                ts = [t_v[bi, i, pl.ds(k * L, L)] for k in range(nk)]
                ssh = _tree_sum([h * h for h in hs])
                sst = _tree_sum([t * t for t in ts])
                ihv = _rsqrt(jnp.maximum(_lane_sum(ssh), NORM_EPS2))
                itv = _rsqrt(jnp.maximum(_lane_sum(sst), NORM_EPS2))
                return hs, ts, ihv, itv

            def score_phase(i, hs, ts, ihv, itv, merged):
                terms = []
                for k in range(nk):
                    rhk = rh_v[bi, i, pl.ds(k * L, L)]
                    rtk = rt_v[bi, i, pl.ds(k * L, L)]
                    terms.append(jnp.abs(hs[k] * ihv * rhk - ts[k] * itv * rtk))
                score = GAMMA - _lane_sum(_tree_sum(terms))
                j = lax.bitwise_and(i, L - 1)
                merged = jnp.where(lane_iota == j, score, merged)

                @pl.when(j == L - 1)
                def _():
                    out_v[lax.shift_right_logical(obase + i, 4), :] = merged

                return merged

            hs0, ts0, ihv0, itv0 = norm_phase(0)
            carry0 = (*hs0, *ts0, ihv0, itv0, jnp.zeros((L,), jnp.float32))

            def body(i, carry):
                hs, ts = list(carry[:nk]), list(carry[nk:2 * nk])
                ihv, itv, merged = carry[2 * nk], carry[2 * nk + 1], carry[2 * nk + 2]
                nxt = jnp.minimum(i + 1, CHUNK - 1)
                hs1, ts1, ihv1, itv1 = norm_phase(nxt)
                merged = score_phase(i, hs, ts, ihv, itv, merged)
                return (*hs1, *ts1, ihv1, itv1, merged)

            lax.fori_loop(0, CHUNK, body, carry0)

        pending = fire(0, 0)
        for c in range(nchunk):
            nxt = fire(c + 1, (c + 1) & 1) if c + 1 < nchunk else None
            for hnd in pending:
                hnd.wait()
            chunk_compute(c, c & 1)
            pending = nxt

        pltpu.sync_copy(out_v, out_hbm.at[pl.ds(wid * (npw // L), npw // L)])

    return rgcn_sc


def kernel(sample, entity_embedding, relation_embedding_head, relation_embedding_tail):
    h_idx = sample[:, 0]
    r_idx = sample[:, 1]
    t_idx = sample[:, 2]
    out = _make_sc_kernel(sample.shape[0])(
        h_idx, r_idx, t_idx,
        entity_embedding, relation_embedding_head, relation_embedding_tail)
    return out.reshape(sample.shape[0])
